# baseline (speedup 1.0000x reference)
import jax
import jax.numpy as jnp
from jax.experimental import pallas as pl
from jax.experimental.pallas import tpu as pltpu

_BR = 512


def _prep_kernel(x_ref, y_ref, wfc_ref, bfc_ref, w1_ref, s1_ref):
    y_new = (
        jnp.dot(y_ref[:], wfc_ref[:], preferred_element_type=jnp.float32)
        + bfc_ref[:]
    )
    x_star = jnp.concatenate([x_ref[:], y_new], axis=0)
    s1_ref[:] = jnp.dot(x_star, w1_ref[:], preferred_element_type=jnp.float32)


def _xe_kernel(e_ref, s1_ref, b1_ref, xe_ref):
    acc = jnp.dot(e_ref[:], s1_ref[:], preferred_element_type=jnp.float32)
    xe_ref[:] = jnp.maximum(acc + b1_ref[:], 0.0)


def _s2_kernel(xe_ref, w2_ref, s2_ref):
    s2_ref[:] = jnp.dot(xe_ref[:], w2_ref[:], preferred_element_type=jnp.float32)


def _out_kernel(a_ref, s2_ref, b2_ref, o_ref):
    acc = jnp.dot(a_ref[:], s2_ref[:], preferred_element_type=jnp.float32)
    o_ref[:] = jax.nn.sigmoid(acc + b2_ref[:])


def kernel(Y_embedding, X, E_tilde, A_tilde, W_fc, b_fc, W1, b1, W2, b2):
    m = E_tilde.shape[0]
    nfeat = X.shape[1]
    nhid = W1.shape[1]
    nclass = W2.shape[1]
    f32 = jnp.float32

    bfc2 = b_fc.reshape(1, nfeat)
    b1_2 = b1.reshape(1, nhid)
    b2_2 = b2.reshape(1, nclass)

    s1 = pl.pallas_call(
        _prep_kernel,
        out_shape=jax.ShapeDtypeStruct((m, nhid), f32),
    )(X, Y_embedding, W_fc, bfc2, W1)

    grid = (pl.cdiv(m, _BR),)
    stream_params = pltpu.CompilerParams(dimension_semantics=("parallel",))

    x_embedding = pl.pallas_call(
        _xe_kernel,
        grid=grid,
        in_specs=[
            pl.BlockSpec((_BR, m), lambda i: (i, 0)),
            pl.BlockSpec((m, nhid), lambda i: (0, 0)),
            pl.BlockSpec((1, nhid), lambda i: (0, 0)),
        ],
        out_specs=pl.BlockSpec((_BR, nhid), lambda i: (i, 0)),
        out_shape=jax.ShapeDtypeStruct((m, nhid), f32),
        compiler_params=stream_params,
    )(E_tilde, s1, b1_2)

    s2 = pl.pallas_call(
        _s2_kernel,
        out_shape=jax.ShapeDtypeStruct((m, nclass), f32),
    )(x_embedding, W2)

    output = pl.pallas_call(
        _out_kernel,
        grid=grid,
        in_specs=[
            pl.BlockSpec((_BR, m), lambda i: (i, 0)),
            pl.BlockSpec((m, nclass), lambda i: (0, 0)),
            pl.BlockSpec((1, nclass), lambda i: (0, 0)),
        ],
        out_specs=pl.BlockSpec((_BR, nclass), lambda i: (i, 0)),
        out_shape=jax.ShapeDtypeStruct((m, nclass), f32),
        compiler_params=stream_params,
    )(A_tilde, s2, b2_2)

    return (output, x_embedding)


# speedup vs baseline: 1.0737x; 1.0737x over previous
import functools

import jax
import jax.numpy as jnp
from jax.experimental import pallas as pl
from jax.experimental.pallas import tpu as pltpu

_BR = 256


def _fused_kernel(
    e_ref, a_ref, x_ref, y_ref, wfc_ref, bfc_ref, w1_ref, b1_ref, w2_ref,
    b2_ref, o_ref, xe_ref, s1_scr, s2_scr, *, grid_half, m
):
    i = pl.program_id(0)

    @pl.when(i == 0)
    def _prep():
        y_new = (
            jnp.dot(y_ref[:], wfc_ref[:], preferred_element_type=jnp.float32)
            + bfc_ref[:]
        )
        x_star = jnp.concatenate([x_ref[:], y_new], axis=0)
        s1_scr[:] = jnp.dot(x_star, w1_ref[:], preferred_element_type=jnp.float32)

    @pl.when(i < grid_half)
    def _phase1():
        acc = jnp.dot(e_ref[:], s1_scr[:], preferred_element_type=jnp.float32)
        xe = jnp.maximum(acc + b1_ref[:], 0.0)
        xe_ref[:] = xe
        s2_scr[pl.ds(i * _BR, _BR), :] = jnp.dot(
            xe, w2_ref[:], preferred_element_type=jnp.float32
        )

    @pl.when(i >= grid_half)
    def _phase2():
        acc = jnp.dot(
            a_ref[:], s2_scr[0:m, :], preferred_element_type=jnp.float32
        )
        o_ref[:] = jax.nn.sigmoid(acc + b2_ref[:])


def kernel(Y_embedding, X, E_tilde, A_tilde, W_fc, b_fc, W1, b1, W2, b2):
    m = E_tilde.shape[0]
    n = X.shape[0]
    nfeat = X.shape[1]
    nhid = W1.shape[1]
    nclass = W2.shape[1]
    nhigh = Y_embedding.shape[1]
    l = Y_embedding.shape[0]
    f32 = jnp.float32

    bfc2 = b_fc.reshape(1, nfeat)
    b1_2 = b1.reshape(1, nhid)
    b2_2 = b2.reshape(1, nclass)

    g = pl.cdiv(m, _BR)

    const = lambda i: (0, 0)
    body = functools.partial(_fused_kernel, grid_half=g, m=m)

    output, x_embedding = pl.pallas_call(
        body,
        grid=(2 * g,),
        in_specs=[
            pl.BlockSpec((_BR, m), lambda i: (jnp.minimum(i, g - 1), 0)),
            pl.BlockSpec((_BR, m), lambda i: (jnp.maximum(i - g, 0), 0)),
            pl.BlockSpec((n, nfeat), const),
            pl.BlockSpec((l, nhigh), const),
            pl.BlockSpec((nhigh, nfeat), const),
            pl.BlockSpec((1, nfeat), const),
            pl.BlockSpec((nfeat, nhid), const),
            pl.BlockSpec((1, nhid), const),
            pl.BlockSpec((nhid, nclass), const),
            pl.BlockSpec((1, nclass), const),
        ],
        out_specs=[
            pl.BlockSpec((_BR, nclass), lambda i: (jnp.maximum(i - g, 0), 0)),
            pl.BlockSpec((_BR, nhid), lambda i: (jnp.minimum(i, g - 1), 0)),
        ],
        out_shape=[
            jax.ShapeDtypeStruct((m, nclass), f32),
            jax.ShapeDtypeStruct((m, nhid), f32),
        ],
        scratch_shapes=[
            pltpu.VMEM((m, nhid), f32),
            pltpu.VMEM((g * _BR, nclass), f32),
        ],
        compiler_params=pltpu.CompilerParams(
            dimension_semantics=("arbitrary",)
        ),
    )(E_tilde, A_tilde, X, Y_embedding, W_fc, bfc2, W1, b1_2, W2, b2_2)

    return (output, x_embedding)
